# baseline (speedup 1.0000x reference)
import functools

import jax
import jax.numpy as jnp
from jax import lax
from jax.experimental import pallas as pl
from jax.experimental.pallas import tpu as pltpu
from jax.experimental.pallas import tpu_sc as plsc

_V = 1000
_D = 252
_DP = 256
_NI = 4096
_NJ = 26
_NW = 32
_IPW = _NI // _NW
_KH = _IPW // 2
_NPAIR = _NJ // 2

_mesh = plsc.VectorSubcoreMesh(core_axis_name="c", subcore_axis_name="s")


@functools.partial(
    pl.kernel,
    out_type=jax.ShapeDtypeStruct((_NJ, _NI, _D), jnp.float32),
    mesh=_mesh,
    scratch_types=[
        pltpu.VMEM((_NJ, _IPW), jnp.int32),
        pltpu.VMEM((_IPW, _DP), jnp.float32),
        pltpu.VMEM((_IPW, _DP), jnp.float32),
        pltpu.VMEM((_KH, _D), jnp.float32),
        pltpu.VMEM((_KH, _D), jnp.float32),
        pltpu.SemaphoreType.DMA,
        pltpu.SemaphoreType.DMA,
        pltpu.SemaphoreType.DMA,
        pltpu.SemaphoreType.DMA,
    ],
)
def _gather_sc(xt_hbm, di_hbm, out_hbm, idx_v, buf0, buf1, cbuf0, cbuf1,
               sg0, sg1, sw0, sw1):
    wid = lax.axis_index("s") * 2 + lax.axis_index("c")
    i_base = wid * _IPW
    pltpu.sync_copy(xt_hbm.at[:, pl.ds(i_base, _IPW)], idx_v)

    def idx_of(j):
        return idx_v.at[j]

    def out_of(j, s):
        return out_hbm.at[j, pl.ds(i_base + s * _KH, _KH)]

    def compact_half(buf, cbuf, s):
        def row_body(r2, rcarry):
            for u in range(2):
                r = r2 * 2 + u
                b = s * _KH + r
                for k in range(15):
                    cbuf[r, pl.ds(16 * k, 16)] = buf[b, pl.ds(16 * k, 16)]
                cbuf[r, pl.ds(_D - 16, 16)] = buf[b, pl.ds(_D - 16, 16)]
            return rcarry

        lax.fori_loop(0, _KH // 2, row_body, 0)

    def process(buf, j, first):
        @pl.when(jnp.logical_not(first))
        def _wait_w0():
            pltpu.make_async_copy(cbuf0, out_of(j - 1, 0), sw0).wait()

        compact_half(buf, cbuf0, 0)
        pltpu.async_copy(cbuf0, out_of(j, 0), sw0)

        @pl.when(jnp.logical_not(first))
        def _wait_w1():
            pltpu.make_async_copy(cbuf1, out_of(j - 1, 1), sw1).wait()

        compact_half(buf, cbuf1, 1)
        pltpu.async_copy(cbuf1, out_of(j, 1), sw1)

    pltpu.async_copy(di_hbm.at[idx_of(0)], buf0, sg0)

    def pair_body(h, carry):
        j0 = 2 * h

        pltpu.make_async_copy(di_hbm.at[idx_of(j0)], buf0, sg0).wait()
        pltpu.async_copy(di_hbm.at[idx_of(j0 + 1)], buf1, sg1)
        process(buf0, j0, h == 0)

        pltpu.make_async_copy(di_hbm.at[idx_of(j0 + 1)], buf1, sg1).wait()

        @pl.when(h < _NPAIR - 1)
        def _g0():
            pltpu.async_copy(di_hbm.at[idx_of(j0 + 2)], buf0, sg0)

        process(buf1, j0 + 1, False)
        return carry

    lax.fori_loop(0, _NPAIR, pair_body, 0)
    pltpu.make_async_copy(cbuf0, out_of(_NJ - 1, 0), sw0).wait()
    pltpu.make_async_copy(cbuf1, out_of(_NJ - 1, 1), sw1).wait()


def kernel(x, di):
    xt = x.T.astype(jnp.int32)
    di_pad = jnp.pad(di, ((0, 0), (0, _DP - _D)))
    out = _gather_sc(xt, di_pad)
    return out.transpose(1, 0, 2)


# speedup vs baseline: 1.0277x; 1.0277x over previous
import functools

import jax
import jax.numpy as jnp
from jax import lax
from jax.experimental import pallas as pl
from jax.experimental.pallas import tpu as pltpu
from jax.experimental.pallas import tpu_sc as plsc

_V = 1000
_D = 252
_DP = 256
_NI = 4096
_NJ = 26
_NW = 32
_IPW = _NI // _NW
_KH = _IPW // 2
_NPAIR = _NJ // 2

_mesh = plsc.VectorSubcoreMesh(core_axis_name="c", subcore_axis_name="s")


@functools.partial(
    pl.kernel,
    out_type=jax.ShapeDtypeStruct((_NJ, _NI, _D), jnp.float32),
    mesh=_mesh,
    scratch_types=[
        pltpu.VMEM((_NJ, _IPW), jnp.int32),
        pltpu.VMEM((_IPW, _DP), jnp.float32),
        pltpu.VMEM((_IPW, _DP), jnp.float32),
        pltpu.VMEM((_KH, _D), jnp.float32),
        pltpu.VMEM((_KH, _D), jnp.float32),
        pltpu.SemaphoreType.DMA,
        pltpu.SemaphoreType.DMA,
        pltpu.SemaphoreType.DMA,
        pltpu.SemaphoreType.DMA,
    ],
)
def _gather_sc(xt_hbm, di_hbm, out_hbm, idx_v, buf0, buf1, cbuf0, cbuf1,
               sg0, sg1, sw0, sw1):
    wid = lax.axis_index("s") * 2 + lax.axis_index("c")
    i_base = wid * _IPW
    pltpu.sync_copy(xt_hbm.at[:, pl.ds(i_base, _IPW)], idx_v)

    def idx_of(j):
        return idx_v.at[j]

    def out_of(j, s):
        return out_hbm.at[j, pl.ds(i_base + s * _KH, _KH)]

    def compact_half(buf, cbuf, s):
        def row_body(r, rcarry):
            for k in range(15):
                cbuf[r, pl.ds(16 * k, 16)] = buf[s * _KH + r, pl.ds(16 * k, 16)]
            cbuf[r, pl.ds(_D - 16, 16)] = buf[s * _KH + r, pl.ds(_D - 16, 16)]
            return rcarry

        lax.fori_loop(0, _KH, row_body, 0)

    def process(buf, j, first):
        @pl.when(jnp.logical_not(first))
        def _wait_w0():
            pltpu.make_async_copy(cbuf0, out_of(j - 1, 0), sw0).wait()

        compact_half(buf, cbuf0, 0)
        pltpu.async_copy(cbuf0, out_of(j, 0), sw0)

        @pl.when(jnp.logical_not(first))
        def _wait_w1():
            pltpu.make_async_copy(cbuf1, out_of(j - 1, 1), sw1).wait()

        compact_half(buf, cbuf1, 1)
        pltpu.async_copy(cbuf1, out_of(j, 1), sw1)

    pltpu.async_copy(di_hbm.at[idx_of(0)], buf0, sg0)

    def pair_body(h, carry):
        j0 = 2 * h

        pltpu.make_async_copy(di_hbm.at[idx_of(j0)], buf0, sg0).wait()
        pltpu.async_copy(di_hbm.at[idx_of(j0 + 1)], buf1, sg1)
        process(buf0, j0, h == 0)

        @pl.when(h < _NPAIR - 1)
        def _g0():
            pltpu.async_copy(di_hbm.at[idx_of(j0 + 2)], buf0, sg0)

        pltpu.make_async_copy(di_hbm.at[idx_of(j0 + 1)], buf1, sg1).wait()
        process(buf1, j0 + 1, False)
        return carry

    lax.fori_loop(0, _NPAIR, pair_body, 0)
    pltpu.make_async_copy(cbuf0, out_of(_NJ - 1, 0), sw0).wait()
    pltpu.make_async_copy(cbuf1, out_of(_NJ - 1, 1), sw1).wait()


def kernel(x, di):
    xt = x.T.astype(jnp.int32)
    di_pad = jnp.pad(di, ((0, 0), (0, _DP - _D)))
    out = _gather_sc(xt, di_pad)
    return out.transpose(1, 0, 2)
